# baseline (speedup 1.0000x reference)
import jax
import jax.numpy as jnp
from jax import lax
from jax.experimental import pallas as pl
from jax.experimental.pallas import tpu as pltpu
from jax.experimental.pallas import tpu_sc as plsc

_BATCH = 16384
_FEAT = 128
_LANES = 16
_NW = 32
_BPW = _BATCH // _NW
_CHUNK = 128
_NCHUNK = _BPW // _CHUNK
_NACC = _FEAT // _LANES
_UNROLL = 4


def _body(feat_hbm, lab_hbm, cent_hbm, out_hbm, idx_v, rows_v, feat_v,
          acc_v, gsem, fsem):
    wid = lax.axis_index("s") * 2 + lax.axis_index("c")
    base = wid * _BPW

    def issue(c):
        b = c % 2
        row0 = base + c * _CHUNK
        f = pltpu.async_copy(feat_hbm.at[pl.ds(row0, _CHUNK)],
                             feat_v.at[b], fsem)
        pltpu.sync_copy(lab_hbm.at[pl.ds(row0, _CHUNK)], idx_v.at[b])
        g = pltpu.async_copy(cent_hbm.at[idx_v.at[b]], rows_v.at[b], gsem)
        return g, f

    pend = issue(0)
    accs = (jnp.zeros((_LANES,), jnp.float32),) * _NACC
    for c in range(_NCHUNK):
        g, f = pend
        if c + 1 < _NCHUNK:
            pend = issue(c + 1)
        g.wait()
        f.wait()
        b = c % 2
        rows_b = rows_v.at[b]
        feat_b = feat_v.at[b]

        def row_body(i, a, rows_b=rows_b, feat_b=feat_b):
            r = i * _UNROLL
            out = list(a)
            for rr in range(_UNROLL):
                for d in range(_NACC):
                    fv = feat_b[r + rr, pl.ds(d * _LANES, _LANES)]
                    gv = rows_b[r + rr, pl.ds(d * _LANES, _LANES)]
                    df = fv - gv
                    out[d] = out[d] + df * df
            return tuple(out)

        accs = lax.fori_loop(0, _CHUNK // _UNROLL, row_body, accs)

    total = accs[0]
    for d in range(1, _NACC):
        total = total + accs[d]
    acc_v[...] = total
    pltpu.sync_copy(acc_v, out_hbm.at[wid])


@jax.jit
def kernel(features, labels, centers):
    mesh = plsc.VectorSubcoreMesh(core_axis_name="c", subcore_axis_name="s")
    partials = pl.kernel(
        _body,
        out_type=jax.ShapeDtypeStruct((_NW, _LANES), jnp.float32),
        mesh=mesh,
        scratch_types=[
            pltpu.VMEM((2, _CHUNK), jnp.int32),
            pltpu.VMEM((2, _CHUNK, _FEAT), jnp.float32),
            pltpu.VMEM((2, _CHUNK, _FEAT), jnp.float32),
            pltpu.VMEM((_LANES,), jnp.float32),
            pltpu.SemaphoreType.DMA,
            pltpu.SemaphoreType.DMA,
        ],
    )(features, labels.astype(jnp.int32), centers)
    return jnp.sum(partials) / (2.0 * features.shape[0])


# speedup vs baseline: 1.0451x; 1.0451x over previous
import jax
import jax.numpy as jnp
from jax import lax
from jax.experimental import pallas as pl
from jax.experimental.pallas import tpu as pltpu
from jax.experimental.pallas import tpu_sc as plsc

_BATCH = 16384
_FEAT = 128
_LANES = 16
_NW = 32
_BPW = _BATCH // _NW
_CHUNK = 128
_NCHUNK = _BPW // _CHUNK
_NACC = _FEAT // _LANES
_UNROLL = 2


def _body(feat_hbm, lab_hbm, cent_hbm, out_hbm, idx_v, rows_v, feat_v,
          acc_v, gsem, fsem):
    wid = lax.axis_index("s") * 2 + lax.axis_index("c")
    base = wid * _BPW

    def issue(c):
        b = c % 2
        row0 = base + c * _CHUNK
        f = pltpu.async_copy(feat_hbm.at[pl.ds(row0, _CHUNK)],
                             feat_v.at[b], fsem)
        pltpu.sync_copy(lab_hbm.at[pl.ds(row0, _CHUNK)], idx_v.at[b])
        g = pltpu.async_copy(cent_hbm.at[idx_v.at[b]], rows_v.at[b], gsem)
        return g, f

    pend = issue(0)
    accs = (jnp.zeros((_LANES,), jnp.float32),) * _NACC
    for c in range(_NCHUNK):
        g, f = pend
        if c + 1 < _NCHUNK:
            pend = issue(c + 1)
        g.wait()
        f.wait()
        b = c % 2
        rows_b = rows_v.at[b]
        feat_b = feat_v.at[b]

        def row_body(i, a, rows_b=rows_b, feat_b=feat_b):
            r = i * _UNROLL
            out = list(a)
            for rr in range(_UNROLL):
                for d in range(_NACC):
                    fv = feat_b[r + rr, pl.ds(d * _LANES, _LANES)]
                    gv = rows_b[r + rr, pl.ds(d * _LANES, _LANES)]
                    df = fv - gv
                    out[d] = out[d] + df * df
            return tuple(out)

        accs = lax.fori_loop(0, _CHUNK // _UNROLL, row_body, accs)

    total = accs[0]
    for d in range(1, _NACC):
        total = total + accs[d]
    acc_v[...] = total
    pltpu.sync_copy(acc_v, out_hbm.at[wid])


@jax.jit
def kernel(features, labels, centers):
    mesh = plsc.VectorSubcoreMesh(core_axis_name="c", subcore_axis_name="s")
    partials = pl.kernel(
        _body,
        out_type=jax.ShapeDtypeStruct((_NW, _LANES), jnp.float32),
        mesh=mesh,
        scratch_types=[
            pltpu.VMEM((2, _CHUNK), jnp.int32),
            pltpu.VMEM((2, _CHUNK, _FEAT), jnp.float32),
            pltpu.VMEM((2, _CHUNK, _FEAT), jnp.float32),
            pltpu.VMEM((_LANES,), jnp.float32),
            pltpu.SemaphoreType.DMA,
            pltpu.SemaphoreType.DMA,
        ],
    )(features, labels.astype(jnp.int32), centers)
    return jnp.sum(partials) / (2.0 * features.shape[0])


# speedup vs baseline: 1.0657x; 1.0198x over previous
import jax
import jax.numpy as jnp
from jax import lax
from jax.experimental import pallas as pl
from jax.experimental.pallas import tpu as pltpu
from jax.experimental.pallas import tpu_sc as plsc

_BATCH = 16384
_FEAT = 128
_LANES = 16
_NW = 32
_BPW = _BATCH // _NW
_CHUNK = 128
_NCHUNK = _BPW // _CHUNK
_NACC = _FEAT // _LANES
_UNROLL = 2
_NBUF = 3


def _body(feat_hbm, lab_hbm, cent_hbm, out_hbm, idx_v, rows_v, feat_v,
          acc_v, gsem, fsem):
    wid = lax.axis_index("s") * 2 + lax.axis_index("c")
    base = wid * _BPW
    pltpu.sync_copy(lab_hbm.at[pl.ds(base, _BPW)], idx_v)

    def issue(c):
        b = c % _NBUF
        row0 = base + c * _CHUNK
        f = pltpu.async_copy(feat_hbm.at[pl.ds(row0, _CHUNK)],
                             feat_v.at[b], fsem)
        g = pltpu.async_copy(cent_hbm.at[idx_v.at[pl.ds(c * _CHUNK, _CHUNK)]],
                             rows_v.at[b], gsem)
        return g, f

    pend = [issue(c) for c in range(_NBUF - 1)]
    accs = (jnp.zeros((_LANES,), jnp.float32),) * _NACC
    for c in range(_NCHUNK):
        g, f = pend.pop(0)
        if c + _NBUF - 1 < _NCHUNK:
            pend.append(issue(c + _NBUF - 1))
        g.wait()
        f.wait()
        b = c % _NBUF
        rows_b = rows_v.at[b]
        feat_b = feat_v.at[b]

        def row_body(i, a, rows_b=rows_b, feat_b=feat_b):
            r = i * _UNROLL
            out = list(a)
            for rr in range(_UNROLL):
                for d in range(_NACC):
                    fv = feat_b[r + rr, pl.ds(d * _LANES, _LANES)]
                    gv = rows_b[r + rr, pl.ds(d * _LANES, _LANES)]
                    df = fv - gv
                    out[d] = out[d] + df * df
            return tuple(out)

        accs = lax.fori_loop(0, _CHUNK // _UNROLL, row_body, accs)

    total = accs[0]
    for d in range(1, _NACC):
        total = total + accs[d]
    acc_v[...] = total
    pltpu.sync_copy(acc_v, out_hbm.at[wid])


@jax.jit
def kernel(features, labels, centers):
    mesh = plsc.VectorSubcoreMesh(core_axis_name="c", subcore_axis_name="s")
    partials = pl.kernel(
        _body,
        out_type=jax.ShapeDtypeStruct((_NW, _LANES), jnp.float32),
        mesh=mesh,
        scratch_types=[
            pltpu.VMEM((_BPW,), jnp.int32),
            pltpu.VMEM((_NBUF, _CHUNK, _FEAT), jnp.float32),
            pltpu.VMEM((_NBUF, _CHUNK, _FEAT), jnp.float32),
            pltpu.VMEM((_LANES,), jnp.float32),
            pltpu.SemaphoreType.DMA,
            pltpu.SemaphoreType.DMA,
        ],
    )(features, labels.astype(jnp.int32), centers)
    return jnp.sum(partials) / (2.0 * features.shape[0])


# speedup vs baseline: 1.0714x; 1.0053x over previous
import jax
import jax.numpy as jnp
from jax import lax
from jax.experimental import pallas as pl
from jax.experimental.pallas import tpu as pltpu
from jax.experimental.pallas import tpu_sc as plsc

_BATCH = 16384
_FEAT = 128
_LANES = 16
_NW = 32
_BPW = _BATCH // _NW
_CHUNK = 64
_NCHUNK = _BPW // _CHUNK
_NACC = _FEAT // _LANES
_UNROLL = 2
_NBUF = 4


def _body(feat_hbm, lab_hbm, cent_hbm, out_hbm, idx_v, rows_v, feat_v,
          acc_v, gsem, fsem):
    wid = lax.axis_index("s") * 2 + lax.axis_index("c")
    base = wid * _BPW
    pltpu.sync_copy(lab_hbm.at[pl.ds(base, _BPW)], idx_v)

    def issue(c):
        b = c % _NBUF
        row0 = base + c * _CHUNK
        f = pltpu.async_copy(feat_hbm.at[pl.ds(row0, _CHUNK)],
                             feat_v.at[b], fsem)
        g = pltpu.async_copy(cent_hbm.at[idx_v.at[pl.ds(c * _CHUNK, _CHUNK)]],
                             rows_v.at[b], gsem)
        return g, f

    pend = [issue(c) for c in range(_NBUF - 1)]
    accs = (jnp.zeros((_LANES,), jnp.float32),) * _NACC
    for c in range(_NCHUNK):
        g, f = pend.pop(0)
        if c + _NBUF - 1 < _NCHUNK:
            pend.append(issue(c + _NBUF - 1))
        g.wait()
        f.wait()
        b = c % _NBUF
        rows_b = rows_v.at[b]
        feat_b = feat_v.at[b]

        def row_body(i, a, rows_b=rows_b, feat_b=feat_b):
            r = i * _UNROLL
            out = list(a)
            for rr in range(_UNROLL):
                for d in range(_NACC):
                    fv = feat_b[r + rr, pl.ds(d * _LANES, _LANES)]
                    gv = rows_b[r + rr, pl.ds(d * _LANES, _LANES)]
                    df = fv - gv
                    out[d] = out[d] + df * df
            return tuple(out)

        accs = lax.fori_loop(0, _CHUNK // _UNROLL, row_body, accs)

    total = accs[0]
    for d in range(1, _NACC):
        total = total + accs[d]
    acc_v[...] = total
    pltpu.sync_copy(acc_v, out_hbm.at[wid])


@jax.jit
def kernel(features, labels, centers):
    mesh = plsc.VectorSubcoreMesh(core_axis_name="c", subcore_axis_name="s")
    partials = pl.kernel(
        _body,
        out_type=jax.ShapeDtypeStruct((_NW, _LANES), jnp.float32),
        mesh=mesh,
        scratch_types=[
            pltpu.VMEM((_BPW,), jnp.int32),
            pltpu.VMEM((_NBUF, _CHUNK, _FEAT), jnp.float32),
            pltpu.VMEM((_NBUF, _CHUNK, _FEAT), jnp.float32),
            pltpu.VMEM((_LANES,), jnp.float32),
            pltpu.SemaphoreType.DMA,
            pltpu.SemaphoreType.DMA,
        ],
    )(features, labels.astype(jnp.int32), centers)
    return jnp.sum(partials) / (2.0 * features.shape[0])
